# baseline (speedup 1.0000x reference)
import functools

import jax
import jax.numpy as jnp
from jax import lax
from jax.experimental import pallas as pl
from jax.experimental.pallas import tpu as pltpu
from jax.experimental.pallas import tpu_sc as plsc

N = 1048576
NCELL = 2097152
NC = 2
NS = 16
NW = NC * NS
L = 16
CHUNK = N // NW
PC = 2048
HALF = NCELL // NC
SLICE = HALF // NS
NB = 2048
CPB = NCELL // NB
RANGE = NCELL // NW
KC = 2048

_params = pltpu.CompilerParams(needs_layout_passes=False)


def _iota():
    return lax.iota(jnp.int32, L)


def _full(v):
    return jnp.full((L,), v, jnp.int32)


@functools.cache
def _build():
    mesh = plsc.VectorSubcoreMesh(core_axis_name="c", subcore_axis_name="s")

    @functools.partial(
        pl.kernel, mesh=mesh, compiler_params=_params,
        out_type=(jax.ShapeDtypeStruct((N,), jnp.int32),
                  jax.ShapeDtypeStruct((NW, NB), jnp.int32)),
        scratch_types=[pltpu.VMEM((PC * 3,), jnp.float32),
                       pltpu.VMEM((PC,), jnp.int32),
                       pltpu.VMEM((NB,), jnp.int32)],
    )
    def k1a(pos_hbm, hash_hbm, grid_hbm, posv, hashv, histv):
        c = lax.axis_index("c")
        s = lax.axis_index("s")
        wid = c * NS + s
        base = wid * CHUNK

        def zero_body(i, _):
            histv[pl.ds(i * L, L)] = jnp.zeros((L,), jnp.int32)
            return 0
        lax.fori_loop(0, NB // L, zero_body, 0)

        def sub_body(sub, _):
            off = base + sub * PC
            pltpu.sync_copy(pos_hbm.at[pl.ds(off * 3, PC * 3)], posv)

            def vec_body(i, _):
                ri = _full(3 * i * L) + _iota() * 3
                x = plsc.load_gather(posv, [ri])
                y = plsc.load_gather(posv, [ri + 1])
                z = plsc.load_gather(posv, [ri + 2])
                sc = jnp.float32(128.0)
                xi = jnp.minimum((x * sc).astype(jnp.int32), 127)
                yi = jnp.minimum((y * sc).astype(jnp.int32), 127)
                zi = jnp.minimum((z * sc).astype(jnp.int32), 127)
                h = jnp.bitwise_or(
                    jnp.bitwise_or(lax.shift_left(xi, 14),
                                   lax.shift_left(yi, 7)), zi)
                hashv[pl.ds(i * L, L)] = h
                b = lax.shift_right_logical(h, 10)
                occ, last = plsc.scan_count(b)
                cur = plsc.load_gather(histv, [b])
                plsc.store_scatter(histv, [b], cur + occ, mask=last)
                return 0
            lax.fori_loop(0, PC // L, vec_body, 0)
            pltpu.sync_copy(hashv, hash_hbm.at[pl.ds(off, PC)])
            return 0
        lax.fori_loop(0, CHUNK // PC, sub_body, 0)
        pltpu.sync_copy(histv, grid_hbm.at[wid])

    @functools.partial(
        pl.kernel, mesh=mesh, compiler_params=_params,
        out_type=(jax.ShapeDtypeStruct((NCELL,), jnp.int32),
                  jax.ShapeDtypeStruct((NW, L), jnp.int32)),
        scratch_types=[pltpu.VMEM_SHARED((HALF,), jnp.int32),
                       pltpu.VMEM((PC,), jnp.int32),
                       pltpu.VMEM((PC // 128, 128), jnp.int32),
                       pltpu.VMEM((PC // 128, 128), jnp.int32),
                       pltpu.VMEM((8192,), jnp.int32),
                       pltpu.VMEM((L,), jnp.int32)],
    )
    def k1b(hash_hbm, counts_hbm, sums_hbm, spc, hv, idx2, upd2, zv, sumv):
        c = lax.axis_index("c")
        s = lax.axis_index("s")
        wid = c * NS + s

        def zero_body(i, _):
            zv[pl.ds(i * L, L)] = jnp.zeros((L,), jnp.int32)
            return 0
        lax.fori_loop(0, 8192 // L, zero_body, 0)

        def zdma_body(j, _):
            pltpu.sync_copy(zv, spc.at[pl.ds(s * SLICE + j * 8192, 8192)])
            return 0
        lax.fori_loop(0, SLICE // 8192, zdma_body, 0)
        plsc.subcore_barrier()

        pts = N // NS

        def sub_body(sub, _):
            pltpu.sync_copy(hash_hbm.at[pl.ds(s * pts + sub * PC, PC)], hv)

            def vec_body(i, _):
                h = hv[pl.ds(i * L, L)]
                lo = jnp.bitwise_and(h, HALF - 1)
                mine = lax.shift_right_logical(h, 20) == c
                row = i // 8
                col = (i % 8) * L
                idx2[row, pl.ds(col, L)] = lo
                upd2[row, pl.ds(col, L)] = jnp.where(mine, 1, 0).astype(jnp.int32)
                return 0
            lax.fori_loop(0, PC // L, vec_body, 0)

            def sdma_body(j, _):
                pltpu.sync_copy(upd2.at[j], spc.at[idx2.at[j]], add=True)
                return 0
            lax.fori_loop(0, PC // 128, sdma_body, 0, unroll=True)
            return 0
        lax.fori_loop(0, pts // PC, sub_body, 0)
        plsc.subcore_barrier()

        gbase = c * HALF + s * SLICE

        def out_body(k, acc):
            pltpu.sync_copy(spc.at[pl.ds(s * SLICE + k * 8192, 8192)], zv)

            def acc_body(i, a):
                return a + zv[pl.ds(i * L, L)]
            acc = lax.fori_loop(0, 8192 // L, acc_body, acc)
            pltpu.sync_copy(zv, counts_hbm.at[pl.ds(gbase + k * 8192, 8192)])
            return acc
        acc = lax.fori_loop(0, SLICE // 8192, out_body,
                            jnp.zeros((L,), jnp.int32))
        sumv[...] = acc
        pltpu.sync_copy(sumv, sums_hbm.at[wid])

    @functools.partial(
        pl.kernel, mesh=mesh, compiler_params=_params,
        out_type=jax.ShapeDtypeStruct((NCELL,), jnp.int32),
        scratch_types=[pltpu.VMEM((8192,), jnp.int32),
                       pltpu.VMEM((NW, L), jnp.int32)],
    )
    def k3(counts_hbm, sums_hbm, starts_hbm, zv, sumsv):
        c = lax.axis_index("c")
        s = lax.axis_index("s")
        wid = c * NS + s
        pltpu.sync_copy(sums_hbm, sumsv)

        def pre_body(w, a):
            r = sumsv[w, :]
            return a + jnp.where(_full(w) < _full(wid), r, 0)
        acc = lax.fori_loop(0, NW, pre_body, jnp.zeros((L,), jnp.int32))
        off0 = jnp.sum(acc)
        gbase = wid * SLICE

        def chunk_body(k, off):
            pltpu.sync_copy(counts_hbm.at[pl.ds(gbase + k * 8192, 8192)], zv)

            def scan_body(i, o):
                x = zv[pl.ds(i * L, L)]
                inc = plsc.cumsum(x)
                zv[pl.ds(i * L, L)] = inc - x + o
                return o + jnp.sum(x)
            off = lax.fori_loop(0, 8192 // L, scan_body, off)
            pltpu.sync_copy(zv, starts_hbm.at[pl.ds(gbase + k * 8192, 8192)])
            return off
        lax.fori_loop(0, SLICE // 8192, chunk_body, off0)

    @functools.partial(
        pl.kernel, mesh=mesh, compiler_params=_params,
        out_type=(jax.ShapeDtypeStruct((N + KC,), jnp.int32),
                  jax.ShapeDtypeStruct((N + KC,), jnp.int32)),
        scratch_types=[pltpu.VMEM((NB,), jnp.int32),
                       pltpu.VMEM((NB,), jnp.int32),
                       pltpu.VMEM((NB,), jnp.int32),
                       pltpu.VMEM((NB,), jnp.int32),
                       pltpu.VMEM((PC,), jnp.int32),
                       pltpu.VMEM((PC,), jnp.int32),
                       pltpu.VMEM((PC,), jnp.int32),
                       pltpu.VMEM((PC,), jnp.int32),
                       pltpu.SemaphoreType.DMA],
    )
    def k4(hash_hbm, grid_hbm, keys_hbm, vals_hbm, rowv, totv, prefv, curv,
           hv, pos2, key2, val2, sem):
        c = lax.axis_index("c")
        s = lax.axis_index("s")
        wid = c * NS + s
        base = wid * CHUNK

        def zero_body(i, _):
            totv[pl.ds(i * L, L)] = jnp.zeros((L,), jnp.int32)
            prefv[pl.ds(i * L, L)] = jnp.zeros((L,), jnp.int32)
            return 0
        lax.fori_loop(0, NB // L, zero_body, 0)

        def grid_body(w, _):
            pltpu.sync_copy(grid_hbm.at[w], rowv)

            def add_body(i, _):
                r = rowv[pl.ds(i * L, L)]
                totv[pl.ds(i * L, L)] = totv[pl.ds(i * L, L)] + r
                prefv[pl.ds(i * L, L)] = prefv[pl.ds(i * L, L)] + jnp.where(
                    _full(w) < _full(wid), r, 0)
                return 0
            lax.fori_loop(0, NB // L, add_body, 0)
            return 0
        lax.fori_loop(0, NW, grid_body, 0)

        def scan_body(i, off):
            t = totv[pl.ds(i * L, L)]
            inc = plsc.cumsum(t)
            curv[pl.ds(i * L, L)] = prefv[pl.ds(i * L, L)] + inc - t + off
            return off + jnp.sum(t)
        lax.fori_loop(0, NB // L, scan_body, jnp.int32(0))

        def sub_body(sub, _):
            off = base + sub * PC
            pltpu.sync_copy(hash_hbm.at[pl.ds(off, PC)], hv)

            def vec_body(i, _):
                h = hv[pl.ds(i * L, L)]
                b = lax.shift_right_logical(h, 10)
                occ, last = plsc.scan_count(b)
                cur = plsc.load_gather(curv, [b])
                plsc.store_scatter(curv, [b], cur + occ, mask=last)
                pos2[pl.ds(i * L, L)] = cur + occ - 1
                key2[pl.ds(i * L, L)] = h
                val2[pl.ds(i * L, L)] = _full(off + i * L) + _iota()
                return 0
            lax.fori_loop(0, PC // L, vec_body, 0)

            h1 = pltpu.async_copy(key2, keys_hbm.at[pos2], sem)
            h2 = pltpu.async_copy(val2, vals_hbm.at[pos2], sem)
            h1.wait()
            h2.wait()
            return 0
        lax.fori_loop(0, CHUNK // PC, sub_body, 0)

    @functools.partial(
        pl.kernel, mesh=mesh, compiler_params=_params,
        out_type=jax.ShapeDtypeStruct((N + L,), jnp.int32),
        scratch_types=[pltpu.VMEM((RANGE,), jnp.int32),
                       pltpu.VMEM((L,), jnp.int32),
                       pltpu.VMEM((KC,), jnp.int32),
                       pltpu.VMEM((KC,), jnp.int32),
                       pltpu.VMEM((KC,), jnp.int32),
                       pltpu.VMEM((KC,), jnp.int32),
                       pltpu.SemaphoreType.DMA],
    )
    def k5(keys_hbm, vals_hbm, starts_hbm, pid_hbm, curv, tmpv, keyb, valsb,
           posb2, valb2, sem):
        c = lax.axis_index("c")
        s = lax.axis_index("s")
        wid = c * NS + s

        pltpu.sync_copy(starts_hbm.at[pl.ds(wid * RANGE, RANGE)], curv)
        gstart = curv[pl.ds(0, L)][0]
        off2 = pl.multiple_of(jnp.minimum((wid + 1) * RANGE, NCELL - L), 8)
        pltpu.sync_copy(starts_hbm.at[pl.ds(off2, L)], tmpv)
        end = jnp.where(wid == NW - 1, jnp.int32(N), tmpv[...][0])
        astart = pl.multiple_of(jnp.bitwise_and(gstart, jnp.int32(-8)), 8)
        nch = (end - astart + KC - 1) // KC

        def chunk_body(k, _):
            coff = pl.multiple_of(astart + k * KC, 8)
            h1 = pltpu.async_copy(keys_hbm.at[pl.ds(coff, KC)], keyb, sem)
            h2 = pltpu.async_copy(vals_hbm.at[pl.ds(coff, KC)], valsb, sem)
            h1.wait()
            h2.wait()

            def vec_body(i, _):
                key = keyb[pl.ds(i * L, L)]
                val = valsb[pl.ds(i * L, L)]
                g = _full(coff + i * L) + _iota()
                valid = jnp.logical_and(g >= gstart, g < end)
                cell = jnp.bitwise_and(key, RANGE - 1)
                occ, last = plsc.scan_count(cell, valid)
                cur = plsc.load_gather(curv, [cell])
                plsc.store_scatter(curv, [cell], cur + occ, mask=last)
                pos = jnp.where(valid, cur + occ - 1, _full(N) + _iota())
                posb2[pl.ds(i * L, L)] = pos
                valb2[pl.ds(i * L, L)] = val
                return 0
            lax.fori_loop(0, KC // L, vec_body, 0)
            pltpu.async_copy(valb2, pid_hbm.at[posb2], sem).wait()
            return 0
        lax.fori_loop(0, nch, chunk_body, 0)

    return k1a, k1b, k3, k4, k5


def kernel(position_stack):
    k1a, k1b, k3, k4, k5 = _build()
    hashes, grid = k1a(position_stack.reshape(-1))
    counts, sums = k1b(hashes)
    starts = k3(counts, sums)
    keys, vals = k4(hashes, grid)
    pid_pad = k5(keys, vals, starts)
    return hashes, starts, counts, pid_pad[:N]


# speedup vs baseline: 2.0426x; 2.0426x over previous
import functools

import jax
import jax.numpy as jnp
from jax import lax
from jax.experimental import pallas as pl
from jax.experimental.pallas import tpu as pltpu
from jax.experimental.pallas import tpu_sc as plsc

N = 1048576
NCELL = 2097152
NC = 2
NS = 16
NW = NC * NS
L = 16
CHUNK = N // NW
PC = 2048
HALF = NCELL // NC
SLICE = HALF // NS
NB = 2048
CPB = NCELL // NB
RANGE = NCELL // NW
HC = 4096
NCH = N // HC
OUTCAP = 40960

_params = pltpu.CompilerParams(needs_layout_passes=False)


def _iota():
    return lax.iota(jnp.int32, L)


def _full(v):
    return jnp.full((L,), v, jnp.int32)


@functools.cache
def _build():
    mesh = plsc.VectorSubcoreMesh(core_axis_name="c", subcore_axis_name="s")

    @functools.partial(
        pl.kernel, mesh=mesh, compiler_params=_params,
        out_type=jax.ShapeDtypeStruct((N,), jnp.int32),
        scratch_types=[pltpu.VMEM((PC * 3,), jnp.float32),
                       pltpu.VMEM((PC,), jnp.int32)],
    )
    def k1a(pos_hbm, hash_hbm, posv, hashv):
        c = lax.axis_index("c")
        s = lax.axis_index("s")
        wid = c * NS + s
        base = wid * CHUNK

        def sub_body(sub, _):
            off = base + sub * PC
            pltpu.sync_copy(pos_hbm.at[pl.ds(off * 3, PC * 3)], posv)

            def vec_body(i, _):
                ri = _full(3 * i * L) + _iota() * 3
                x = plsc.load_gather(posv, [ri])
                y = plsc.load_gather(posv, [ri + 1])
                z = plsc.load_gather(posv, [ri + 2])
                sc = jnp.float32(128.0)
                xi = jnp.minimum((x * sc).astype(jnp.int32), 127)
                yi = jnp.minimum((y * sc).astype(jnp.int32), 127)
                zi = jnp.minimum((z * sc).astype(jnp.int32), 127)
                h = jnp.bitwise_or(
                    jnp.bitwise_or(lax.shift_left(xi, 14),
                                   lax.shift_left(yi, 7)), zi)
                hashv[pl.ds(i * L, L)] = h
                return 0
            lax.fori_loop(0, PC // L, vec_body, 0)
            pltpu.sync_copy(hashv, hash_hbm.at[pl.ds(off, PC)])
            return 0
        lax.fori_loop(0, CHUNK // PC, sub_body, 0)

    @functools.partial(
        pl.kernel, mesh=mesh, compiler_params=_params,
        out_type=(jax.ShapeDtypeStruct((NCELL,), jnp.int32),
                  jax.ShapeDtypeStruct((NW, L), jnp.int32)),
        scratch_types=[pltpu.VMEM_SHARED((HALF,), jnp.int32),
                       pltpu.VMEM((PC,), jnp.int32),
                       pltpu.VMEM((PC // 128, 128), jnp.int32),
                       pltpu.VMEM((PC // 128, 128), jnp.int32),
                       pltpu.VMEM((8192,), jnp.int32),
                       pltpu.VMEM((L,), jnp.int32)],
    )
    def k1b(hash_hbm, counts_hbm, sums_hbm, spc, hv, idx2, upd2, zv, sumv):
        c = lax.axis_index("c")
        s = lax.axis_index("s")
        wid = c * NS + s

        def zero_body(i, _):
            zv[pl.ds(i * L, L)] = jnp.zeros((L,), jnp.int32)
            return 0
        lax.fori_loop(0, 8192 // L, zero_body, 0)

        def zdma_body(j, _):
            pltpu.sync_copy(zv, spc.at[pl.ds(s * SLICE + j * 8192, 8192)])
            return 0
        lax.fori_loop(0, SLICE // 8192, zdma_body, 0)
        plsc.subcore_barrier()

        pts = N // NS

        def sub_body(sub, _):
            pltpu.sync_copy(hash_hbm.at[pl.ds(s * pts + sub * PC, PC)], hv)

            def vec_body(i, _):
                h = hv[pl.ds(i * L, L)]
                lo = jnp.bitwise_and(h, HALF - 1)
                mine = lax.shift_right_logical(h, 20) == c
                row = i // 8
                col = (i % 8) * L
                idx2[row, pl.ds(col, L)] = lo
                upd2[row, pl.ds(col, L)] = jnp.where(mine, 1, 0).astype(jnp.int32)
                return 0
            lax.fori_loop(0, PC // L, vec_body, 0)

            def sdma_body(j, _):
                pltpu.sync_copy(upd2.at[j], spc.at[idx2.at[j]], add=True)
                return 0
            lax.fori_loop(0, PC // 128, sdma_body, 0, unroll=True)
            return 0
        lax.fori_loop(0, pts // PC, sub_body, 0)
        plsc.subcore_barrier()

        gbase = c * HALF + s * SLICE

        def out_body(k, acc):
            pltpu.sync_copy(spc.at[pl.ds(s * SLICE + k * 8192, 8192)], zv)

            def acc_body(i, a):
                return a + zv[pl.ds(i * L, L)]
            acc = lax.fori_loop(0, 8192 // L, acc_body, acc)
            pltpu.sync_copy(zv, counts_hbm.at[pl.ds(gbase + k * 8192, 8192)])
            return acc
        acc = lax.fori_loop(0, SLICE // 8192, out_body,
                            jnp.zeros((L,), jnp.int32))
        sumv[...] = acc
        pltpu.sync_copy(sumv, sums_hbm.at[wid])

    @functools.partial(
        pl.kernel, mesh=mesh, compiler_params=_params,
        out_type=jax.ShapeDtypeStruct((NCELL,), jnp.int32),
        scratch_types=[pltpu.VMEM((8192,), jnp.int32),
                       pltpu.VMEM((NW, L), jnp.int32)],
    )
    def k3(counts_hbm, sums_hbm, starts_hbm, zv, sumsv):
        c = lax.axis_index("c")
        s = lax.axis_index("s")
        wid = c * NS + s
        pltpu.sync_copy(sums_hbm, sumsv)

        def pre_body(w, a):
            r = sumsv[w, :]
            return a + jnp.where(_full(w) < _full(wid), r, 0)
        acc = lax.fori_loop(0, NW, pre_body, jnp.zeros((L,), jnp.int32))
        off0 = jnp.sum(acc)
        gbase = wid * SLICE

        def chunk_body(k, off):
            pltpu.sync_copy(counts_hbm.at[pl.ds(gbase + k * 8192, 8192)], zv)

            def scan_body(i, o):
                x = zv[pl.ds(i * L, L)]
                inc = plsc.cumsum(x)
                zv[pl.ds(i * L, L)] = inc - x + o
                return o + jnp.sum(x)
            off = lax.fori_loop(0, 8192 // L, scan_body, off)
            pltpu.sync_copy(zv, starts_hbm.at[pl.ds(gbase + k * 8192, 8192)])
            return off
        lax.fori_loop(0, SLICE // 8192, chunk_body, off0)

    @functools.partial(
        pl.kernel, mesh=mesh, compiler_params=_params,
        out_type=jax.ShapeDtypeStruct((N,), jnp.int32),
        scratch_types=[pltpu.VMEM((RANGE,), jnp.int32),
                       pltpu.VMEM((OUTCAP,), jnp.int32),
                       pltpu.VMEM((HC,), jnp.int32),
                       pltpu.VMEM((HC,), jnp.int32),
                       pltpu.VMEM((L,), jnp.int32),
                       pltpu.VMEM((L,), jnp.int32),
                       pltpu.VMEM((L,), jnp.int32),
                       pltpu.SemaphoreType.DMA,
                       pltpu.SemaphoreType.DMA],
    )
    def k6(hash_hbm, starts_hbm, pid_hbm, curv, outbuf, hb0, hb1, tmpv,
           eposv, evalv, sem0, sem1):
        c = lax.axis_index("c")
        s = lax.axis_index("s")
        wid = c * NS + s

        pltpu.sync_copy(starts_hbm.at[pl.ds(wid * RANGE, RANGE)], curv)
        gstart = curv[pl.ds(0, L)][0]
        off2 = pl.multiple_of(jnp.minimum((wid + 1) * RANGE, NCELL - L), 8)
        pltpu.sync_copy(starts_hbm.at[pl.ds(off2, L)], tmpv)
        end = jnp.where(wid == NW - 1, jnp.int32(N), tmpv[...][0])
        g8 = jnp.bitwise_and(gstart + 7, jnp.int32(-8))
        e8 = jnp.bitwise_and(end, jnp.int32(-8))
        ncap = e8 - g8
        fast = ncap <= OUTCAP
        widv = _full(wid)
        fastv = jnp.full((L,), fast)

        def process(hb, k):
            def vec_body(i, _):
                h = hb[pl.ds(i * L, L)]
                valid = lax.shift_right_logical(h, 16) == widv
                nv = plsc.all_reduce_population_count(valid)

                @pl.when(nv[0] > 0)
                def _():
                    cell = jnp.bitwise_and(h, RANGE - 1)
                    occ, last = plsc.scan_count(cell, valid)
                    cur = plsc.load_gather(curv, [cell])
                    plsc.store_scatter(curv, [cell], cur + occ, mask=last)
                    pos = cur + occ - 1
                    val = _full(k * HC + i * L) + _iota()
                    inb = jnp.logical_and(
                        valid,
                        jnp.logical_and(fastv,
                                        jnp.logical_and(pos >= g8, pos < e8)))
                    plsc.store_scatter(outbuf, [pos - g8], val, mask=inb)
                    edge = jnp.logical_and(valid, jnp.logical_not(inb))
                    ne = plsc.all_reduce_population_count(edge)

                    @pl.when(ne[0] > 0)
                    def _():
                        f = plsc.all_reduce_ffs(edge)
                        eposv[...] = pos
                        evalv[...] = val
                        pfirst = plsc.load_gather(eposv, [f])
                        vfirst = plsc.load_gather(evalv, [f])
                        eposv[...] = jnp.where(edge, pos, pfirst)
                        evalv[...] = jnp.where(edge, val, vfirst)
                        pltpu.sync_copy(evalv, pid_hbm.at[eposv])
                return 0
            lax.fori_loop(0, HC // L, vec_body, 0)

        pltpu.async_copy(hash_hbm.at[pl.ds(0, HC)], hb0, sem0)

        def chunk_body(k, _):
            for par, hb, hbn, semc, semn in ((0, hb0, hb1, sem0, sem1),
                                             (1, hb1, hb0, sem1, sem0)):
                @pl.when(jnp.bitwise_and(k, 1) == par)
                def _():
                    off = pl.multiple_of(k * HC, 8)
                    pltpu.make_async_copy(
                        hash_hbm.at[pl.ds(off, HC)], hb, semc).wait()

                    @pl.when(k + 1 < NCH)
                    def _():
                        offn = pl.multiple_of((k + 1) * HC, 8)
                        pltpu.async_copy(
                            hash_hbm.at[pl.ds(offn, HC)], hbn, semn)
                    process(hb, k)
            return 0
        lax.fori_loop(0, NCH, chunk_body, 0)

        @pl.when(jnp.logical_and(fast, ncap > 0))
        def _():
            nfull = jnp.maximum(ncap, 0) // 8192

            def full_body(j, _):
                o = pl.multiple_of(j * 8192, 8)
                pltpu.sync_copy(outbuf.at[pl.ds(o, 8192)],
                                pid_hbm.at[pl.ds(pl.multiple_of(g8 + o, 8),
                                                 8192)])
                return 0
            lax.fori_loop(0, nfull, full_body, 0)
            rem = jnp.maximum(ncap, 0) - nfull * 8192
            offcur = nfull * 8192
            for size in (4096, 2048, 1024, 512, 256, 128, 64, 32, 16, 8):
                bit = jnp.bitwise_and(rem, size) != 0
                o = pl.multiple_of(offcur, 8)

                @pl.when(bit)
                def _(o=o, size=size):
                    pltpu.sync_copy(
                        outbuf.at[pl.ds(o, size)],
                        pid_hbm.at[pl.ds(pl.multiple_of(g8 + o, 8), size)])
                offcur = offcur + jnp.where(bit, size, 0)

    return k1a, k1b, k3, k6


def kernel(position_stack):
    k1a, k1b, k3, k6 = _build()
    hashes = k1a(position_stack.reshape(-1))
    counts, sums = k1b(hashes)
    starts = k3(counts, sums)
    pid = k6(hashes, starts)
    return hashes, starts, counts, pid


# speedup vs baseline: 2.0449x; 1.0011x over previous
import functools

import jax
import jax.numpy as jnp
from jax import lax
from jax.experimental import pallas as pl
from jax.experimental.pallas import tpu as pltpu
from jax.experimental.pallas import tpu_sc as plsc

N = 1048576
NCELL = 2097152
NC = 2
NS = 16
NW = NC * NS
L = 16
CHUNK = N // NW
PC = 2048
HALF = NCELL // NC
SLICE = HALF // NS
NB = 2048
CPB = NCELL // NB
RANGE = NCELL // NW
HC = 8192
NCH = N // HC
OUTCAP = 36864

_params = pltpu.CompilerParams(needs_layout_passes=False)


def _iota():
    return lax.iota(jnp.int32, L)


def _full(v):
    return jnp.full((L,), v, jnp.int32)


@functools.cache
def _build():
    mesh = plsc.VectorSubcoreMesh(core_axis_name="c", subcore_axis_name="s")

    @functools.partial(
        pl.kernel, mesh=mesh, compiler_params=_params,
        out_type=jax.ShapeDtypeStruct((N,), jnp.int32),
        scratch_types=[pltpu.VMEM((PC * 3,), jnp.float32),
                       pltpu.VMEM((PC,), jnp.int32)],
    )
    def k1a(pos_hbm, hash_hbm, posv, hashv):
        c = lax.axis_index("c")
        s = lax.axis_index("s")
        wid = c * NS + s
        base = wid * CHUNK

        def sub_body(sub, _):
            off = base + sub * PC
            pltpu.sync_copy(pos_hbm.at[pl.ds(off * 3, PC * 3)], posv)

            def vec_body(i, _):
                ri = _full(3 * i * L) + _iota() * 3
                x = plsc.load_gather(posv, [ri])
                y = plsc.load_gather(posv, [ri + 1])
                z = plsc.load_gather(posv, [ri + 2])
                sc = jnp.float32(128.0)
                xi = jnp.minimum((x * sc).astype(jnp.int32), 127)
                yi = jnp.minimum((y * sc).astype(jnp.int32), 127)
                zi = jnp.minimum((z * sc).astype(jnp.int32), 127)
                h = jnp.bitwise_or(
                    jnp.bitwise_or(lax.shift_left(xi, 14),
                                   lax.shift_left(yi, 7)), zi)
                hashv[pl.ds(i * L, L)] = h
                return 0
            lax.fori_loop(0, PC // L, vec_body, 0)
            pltpu.sync_copy(hashv, hash_hbm.at[pl.ds(off, PC)])
            return 0
        lax.fori_loop(0, CHUNK // PC, sub_body, 0)

    @functools.partial(
        pl.kernel, mesh=mesh, compiler_params=_params,
        out_type=(jax.ShapeDtypeStruct((NCELL,), jnp.int32),
                  jax.ShapeDtypeStruct((NW, L), jnp.int32)),
        scratch_types=[pltpu.VMEM_SHARED((HALF,), jnp.int32),
                       pltpu.VMEM((PC,), jnp.int32),
                       pltpu.VMEM((PC // 128, 128), jnp.int32),
                       pltpu.VMEM((PC // 128, 128), jnp.int32),
                       pltpu.VMEM((8192,), jnp.int32),
                       pltpu.VMEM((L,), jnp.int32)],
    )
    def k1b(hash_hbm, counts_hbm, sums_hbm, spc, hv, idx2, upd2, zv, sumv):
        c = lax.axis_index("c")
        s = lax.axis_index("s")
        wid = c * NS + s

        def zero_body(i, _):
            zv[pl.ds(i * L, L)] = jnp.zeros((L,), jnp.int32)
            return 0
        lax.fori_loop(0, 8192 // L, zero_body, 0)

        def zdma_body(j, _):
            pltpu.sync_copy(zv, spc.at[pl.ds(s * SLICE + j * 8192, 8192)])
            return 0
        lax.fori_loop(0, SLICE // 8192, zdma_body, 0)
        plsc.subcore_barrier()

        pts = N // NS

        def sub_body(sub, _):
            pltpu.sync_copy(hash_hbm.at[pl.ds(s * pts + sub * PC, PC)], hv)

            def vec_body(i, _):
                h = hv[pl.ds(i * L, L)]
                lo = jnp.bitwise_and(h, HALF - 1)
                mine = lax.shift_right_logical(h, 20) == c
                row = i // 8
                col = (i % 8) * L
                idx2[row, pl.ds(col, L)] = lo
                upd2[row, pl.ds(col, L)] = jnp.where(mine, 1, 0).astype(jnp.int32)
                return 0
            lax.fori_loop(0, PC // L, vec_body, 0)

            def sdma_body(j, _):
                pltpu.sync_copy(upd2.at[j], spc.at[idx2.at[j]], add=True)
                return 0
            lax.fori_loop(0, PC // 128, sdma_body, 0, unroll=True)
            return 0
        lax.fori_loop(0, pts // PC, sub_body, 0)
        plsc.subcore_barrier()

        gbase = c * HALF + s * SLICE

        def out_body(k, acc):
            pltpu.sync_copy(spc.at[pl.ds(s * SLICE + k * 8192, 8192)], zv)

            def acc_body(i, a):
                return a + zv[pl.ds(i * L, L)]
            acc = lax.fori_loop(0, 8192 // L, acc_body, acc)
            pltpu.sync_copy(zv, counts_hbm.at[pl.ds(gbase + k * 8192, 8192)])
            return acc
        acc = lax.fori_loop(0, SLICE // 8192, out_body,
                            jnp.zeros((L,), jnp.int32))
        sumv[...] = acc
        pltpu.sync_copy(sumv, sums_hbm.at[wid])

    @functools.partial(
        pl.kernel, mesh=mesh, compiler_params=_params,
        out_type=jax.ShapeDtypeStruct((NCELL,), jnp.int32),
        scratch_types=[pltpu.VMEM((8192,), jnp.int32),
                       pltpu.VMEM((NW, L), jnp.int32)],
    )
    def k3(counts_hbm, sums_hbm, starts_hbm, zv, sumsv):
        c = lax.axis_index("c")
        s = lax.axis_index("s")
        wid = c * NS + s
        pltpu.sync_copy(sums_hbm, sumsv)

        def pre_body(w, a):
            r = sumsv[w, :]
            return a + jnp.where(_full(w) < _full(wid), r, 0)
        acc = lax.fori_loop(0, NW, pre_body, jnp.zeros((L,), jnp.int32))
        off0 = jnp.sum(acc)
        gbase = wid * SLICE

        def chunk_body(k, off):
            pltpu.sync_copy(counts_hbm.at[pl.ds(gbase + k * 8192, 8192)], zv)

            def scan_body(i, o):
                x = zv[pl.ds(i * L, L)]
                inc = plsc.cumsum(x)
                zv[pl.ds(i * L, L)] = inc - x + o
                return o + jnp.sum(x)
            off = lax.fori_loop(0, 8192 // L, scan_body, off)
            pltpu.sync_copy(zv, starts_hbm.at[pl.ds(gbase + k * 8192, 8192)])
            return off
        lax.fori_loop(0, SLICE // 8192, chunk_body, off0)

    @functools.partial(
        pl.kernel, mesh=mesh, compiler_params=_params,
        out_type=jax.ShapeDtypeStruct((N,), jnp.int32),
        scratch_types=[pltpu.VMEM((RANGE,), jnp.int32),
                       pltpu.VMEM((OUTCAP,), jnp.int32),
                       pltpu.VMEM((HC,), jnp.int32),
                       pltpu.VMEM((HC,), jnp.int32),
                       pltpu.VMEM((L,), jnp.int32),
                       pltpu.VMEM((L,), jnp.int32),
                       pltpu.VMEM((L,), jnp.int32),
                       pltpu.SemaphoreType.DMA,
                       pltpu.SemaphoreType.DMA],
    )
    def k6(hash_hbm, starts_hbm, pid_hbm, curv, outbuf, hb0, hb1, tmpv,
           eposv, evalv, sem0, sem1):
        c = lax.axis_index("c")
        s = lax.axis_index("s")
        wid = c * NS + s

        pltpu.sync_copy(starts_hbm.at[pl.ds(wid * RANGE, RANGE)], curv)
        gstart = curv[pl.ds(0, L)][0]
        off2 = pl.multiple_of(jnp.minimum((wid + 1) * RANGE, NCELL - L), 8)
        pltpu.sync_copy(starts_hbm.at[pl.ds(off2, L)], tmpv)
        end = jnp.where(wid == NW - 1, jnp.int32(N), tmpv[...][0])
        g8 = jnp.bitwise_and(gstart + 7, jnp.int32(-8))
        e8 = jnp.bitwise_and(end, jnp.int32(-8))
        ncap = e8 - g8
        fast = ncap <= OUTCAP
        widv = _full(wid)
        fastv = jnp.full((L,), fast)

        def process(hb, k):
            def vec_body(i, _):
                h = hb[pl.ds(i * L, L)]
                valid = lax.shift_right_logical(h, 16) == widv
                nv = plsc.all_reduce_population_count(valid)

                @pl.when(nv[0] > 0)
                def _():
                    cell = jnp.bitwise_and(h, RANGE - 1)
                    occ, last = plsc.scan_count(cell, valid)
                    cur = plsc.load_gather(curv, [cell])
                    plsc.store_scatter(curv, [cell], cur + occ, mask=last)
                    pos = cur + occ - 1
                    val = _full(k * HC + i * L) + _iota()
                    inb = jnp.logical_and(
                        valid,
                        jnp.logical_and(fastv,
                                        jnp.logical_and(pos >= g8, pos < e8)))
                    plsc.store_scatter(outbuf, [pos - g8], val, mask=inb)
                    edge = jnp.logical_and(valid, jnp.logical_not(inb))
                    ne = plsc.all_reduce_population_count(edge)

                    @pl.when(ne[0] > 0)
                    def _():
                        f = plsc.all_reduce_ffs(edge)
                        eposv[...] = pos
                        evalv[...] = val
                        pfirst = plsc.load_gather(eposv, [f])
                        vfirst = plsc.load_gather(evalv, [f])
                        eposv[...] = jnp.where(edge, pos, pfirst)
                        evalv[...] = jnp.where(edge, val, vfirst)
                        pltpu.sync_copy(evalv, pid_hbm.at[eposv])
                return 0
            lax.fori_loop(0, HC // L, vec_body, 0, unroll=2)

        pltpu.async_copy(hash_hbm.at[pl.ds(0, HC)], hb0, sem0)

        def chunk_body(k, _):
            for par, hb, hbn, semc, semn in ((0, hb0, hb1, sem0, sem1),
                                             (1, hb1, hb0, sem1, sem0)):
                @pl.when(jnp.bitwise_and(k, 1) == par)
                def _():
                    off = pl.multiple_of(k * HC, 8)
                    pltpu.make_async_copy(
                        hash_hbm.at[pl.ds(off, HC)], hb, semc).wait()

                    @pl.when(k + 1 < NCH)
                    def _():
                        offn = pl.multiple_of((k + 1) * HC, 8)
                        pltpu.async_copy(
                            hash_hbm.at[pl.ds(offn, HC)], hbn, semn)
                    process(hb, k)
            return 0
        lax.fori_loop(0, NCH, chunk_body, 0)

        @pl.when(jnp.logical_and(fast, ncap > 0))
        def _():
            nfull = jnp.maximum(ncap, 0) // 8192

            def full_body(j, _):
                o = pl.multiple_of(j * 8192, 8)
                pltpu.sync_copy(outbuf.at[pl.ds(o, 8192)],
                                pid_hbm.at[pl.ds(pl.multiple_of(g8 + o, 8),
                                                 8192)])
                return 0
            lax.fori_loop(0, nfull, full_body, 0)
            rem = jnp.maximum(ncap, 0) - nfull * 8192
            offcur = nfull * 8192
            for size in (4096, 2048, 1024, 512, 256, 128, 64, 32, 16, 8):
                bit = jnp.bitwise_and(rem, size) != 0
                o = pl.multiple_of(offcur, 8)

                @pl.when(bit)
                def _(o=o, size=size):
                    pltpu.sync_copy(
                        outbuf.at[pl.ds(o, size)],
                        pid_hbm.at[pl.ds(pl.multiple_of(g8 + o, 8), size)])
                offcur = offcur + jnp.where(bit, size, 0)

    return k1a, k1b, k3, k6


def kernel(position_stack):
    k1a, k1b, k3, k6 = _build()
    hashes = k1a(position_stack.reshape(-1))
    counts, sums = k1b(hashes)
    starts = k3(counts, sums)
    pid = k6(hashes, starts)
    return hashes, starts, counts, pid
